# Initial kernel scaffold; baseline (speedup 1.0000x reference)
import jax
import jax.numpy as jnp
from jax.experimental import pallas as pl


def kernel(hidden_state, v_in, g_in, b_in, codebook, v_out, g_out, b_out):
    raise NotImplementedError("write your pallas kernel here")


# speedup vs baseline: 1.8449x; 1.8449x over previous
import jax
import jax.numpy as jnp
from jax.experimental import pallas as pl

B, LATENT, T = 8, 1024, 2048
D, K = 64, 1024
TT = 512
NT = T // TT


def _vq_kernel(x_ref, w_in_ref, b_in_ref, cbn_ref, csq_ref, cb_ref,
               w_out_ref, b_out_ref,
               out_ref, loss_ref, idx_ref, proj_ref):
    t = pl.program_id(1)

    x = x_ref[0]
    p = jax.lax.dot_general(w_in_ref[...], x, (((1,), (0,)), ((), ())))
    p = p + b_in_ref[...]

    norm = jnp.sqrt(jnp.sum(p * p, axis=0, keepdims=True))
    en = p / jnp.maximum(norm, 1e-12)
    l2 = jnp.sum(en * en, axis=0, keepdims=True)
    s = jax.lax.dot_general(cbn_ref[...], en, (((1,), (0,)), ((), ())))
    dist = l2 - 2.0 * s + csq_ref[...]

    m = jnp.min(dist, axis=0, keepdims=True)
    iota = jax.lax.broadcasted_iota(jnp.int32, dist.shape, 0)
    idx = jnp.min(jnp.where(dist == m, iota, K), axis=0)
    idx_ref[0, 0, :] = idx

    oh = (iota == idx[None, :]).astype(jnp.float32)
    q = jax.lax.dot_general(cb_ref[...], oh, (((0,), (0,)), ((), ())),
                            precision=jax.lax.Precision.HIGHEST)

    proj_ref[0] = p

    loss_tile = jnp.sum((p - q) ** 2)
    prev = jnp.where(t == 0, jnp.zeros_like(loss_ref), loss_ref[...])
    loss_ref[...] = prev + loss_tile

    out = jax.lax.dot_general(w_out_ref[...], q, (((1,), (0,)), ((), ())))
    out_ref[0] = out + b_out_ref[...]


def kernel(hidden_state, v_in, g_in, b_in, codebook, v_out, g_out, b_out):
    w_in = v_in * (g_in[:, None] / jnp.sqrt(jnp.sum(v_in * v_in, axis=1, keepdims=True)))
    w_out = v_out * (g_out[:, None] / jnp.sqrt(jnp.sum(v_out * v_out, axis=1, keepdims=True)))
    cbn = codebook / jnp.clip(jnp.linalg.norm(codebook, axis=1, keepdims=True), 1e-12)
    csq = jnp.sum(cbn ** 2, axis=1, keepdims=True)

    out_shapes = (
        jax.ShapeDtypeStruct((B, LATENT, T), jnp.float32),
        jax.ShapeDtypeStruct((B, 1, 1), jnp.float32),
        jax.ShapeDtypeStruct((B * NT, 1, TT), jnp.int32),
        jax.ShapeDtypeStruct((B, D, T), jnp.float32),
    )
    out, loss_sum, idx, proj = pl.pallas_call(
        _vq_kernel,
        grid=(B, NT),
        in_specs=[
            pl.BlockSpec((1, LATENT, TT), lambda b, t: (b, 0, t)),
            pl.BlockSpec((D, LATENT), lambda b, t: (0, 0)),
            pl.BlockSpec((D, 1), lambda b, t: (0, 0)),
            pl.BlockSpec((K, D), lambda b, t: (0, 0)),
            pl.BlockSpec((K, 1), lambda b, t: (0, 0)),
            pl.BlockSpec((K, D), lambda b, t: (0, 0)),
            pl.BlockSpec((LATENT, D), lambda b, t: (0, 0)),
            pl.BlockSpec((LATENT, 1), lambda b, t: (0, 0)),
        ],
        out_specs=(
            pl.BlockSpec((1, LATENT, TT), lambda b, t: (b, 0, t)),
            pl.BlockSpec((1, 1, 1), lambda b, t: (b, 0, 0)),
            pl.BlockSpec((1, 1, TT), lambda b, t: (b * NT + t, 0, 0)),
            pl.BlockSpec((1, D, TT), lambda b, t: (b, 0, t)),
        ),
        out_shape=out_shapes,
    )(hidden_state, w_in, b_in[:, None], cbn, csq, codebook,
      w_out, b_out[:, None])

    loss = loss_sum[:, 0, 0] / (D * T)
    indices = idx.reshape(B, T)
    return (out, loss, loss, indices, proj)


# speedup vs baseline: 2.3864x; 1.2935x over previous
import jax
import jax.numpy as jnp
from jax.experimental import pallas as pl

B, LATENT, T = 8, 1024, 2048
D, K = 64, 1024
TT = 512
NT = T // TT


def _vq_kernel(x_ref, w_in_ref, b_in_ref, cbn_ref, csq_ref, cb_ref,
               w_out_ref, b_out_ref,
               out_ref, loss_ref, idx_ref, proj_ref):
    t = pl.program_id(1)

    x = x_ref[0]
    p = jax.lax.dot_general(w_in_ref[...], x, (((1,), (0,)), ((), ())))
    p = p + b_in_ref[...]

    norm = jnp.sqrt(jnp.sum(p * p, axis=0, keepdims=True))
    en = p / jnp.maximum(norm, 1e-12)
    l2 = jnp.sum(en * en, axis=0, keepdims=True)
    s = jax.lax.dot_general(cbn_ref[...], en, (((1,), (0,)), ((), ())))
    dist = l2 - 2.0 * s + csq_ref[...]

    m = jnp.min(dist, axis=0, keepdims=True)
    iota = jax.lax.broadcasted_iota(jnp.int32, dist.shape, 0)
    idx = jnp.min(jnp.where(dist == m, iota, K), axis=0)
    idx_ref[0, 0, :] = idx

    oh = (iota == idx[None, :]).astype(jnp.float32)
    q = jax.lax.dot_general(cb_ref[...], oh, (((0,), (0,)), ((), ())))

    proj_ref[0] = p

    loss_tile = jnp.sum((p - q) ** 2)
    prev = jnp.where(t == 0, jnp.zeros_like(loss_ref), loss_ref[...])
    loss_ref[...] = prev + loss_tile

    out = jax.lax.dot_general(w_out_ref[...], q, (((1,), (0,)), ((), ())))
    out_ref[0] = out + b_out_ref[...]


def kernel(hidden_state, v_in, g_in, b_in, codebook, v_out, g_out, b_out):
    w_in = v_in * (g_in[:, None] / jnp.sqrt(jnp.sum(v_in * v_in, axis=1, keepdims=True)))
    w_out = v_out * (g_out[:, None] / jnp.sqrt(jnp.sum(v_out * v_out, axis=1, keepdims=True)))
    cbn = codebook / jnp.clip(jnp.linalg.norm(codebook, axis=1, keepdims=True), 1e-12)
    csq = jnp.sum(cbn ** 2, axis=1, keepdims=True)

    out_shapes = (
        jax.ShapeDtypeStruct((B, LATENT, T), jnp.float32),
        jax.ShapeDtypeStruct((B, 1, 1), jnp.float32),
        jax.ShapeDtypeStruct((B * NT, 1, TT), jnp.int32),
        jax.ShapeDtypeStruct((B, D, T), jnp.float32),
    )
    out, loss_sum, idx, proj = pl.pallas_call(
        _vq_kernel,
        grid=(B, NT),
        in_specs=[
            pl.BlockSpec((1, LATENT, TT), lambda b, t: (b, 0, t)),
            pl.BlockSpec((D, LATENT), lambda b, t: (0, 0)),
            pl.BlockSpec((D, 1), lambda b, t: (0, 0)),
            pl.BlockSpec((K, D), lambda b, t: (0, 0)),
            pl.BlockSpec((K, 1), lambda b, t: (0, 0)),
            pl.BlockSpec((K, D), lambda b, t: (0, 0)),
            pl.BlockSpec((LATENT, D), lambda b, t: (0, 0)),
            pl.BlockSpec((LATENT, 1), lambda b, t: (0, 0)),
        ],
        out_specs=(
            pl.BlockSpec((1, LATENT, TT), lambda b, t: (b, 0, t)),
            pl.BlockSpec((1, 1, 1), lambda b, t: (b, 0, 0)),
            pl.BlockSpec((1, 1, TT), lambda b, t: (b * NT + t, 0, 0)),
            pl.BlockSpec((1, D, TT), lambda b, t: (b, 0, t)),
        ),
        out_shape=out_shapes,
    )(hidden_state, w_in, b_in[:, None], cbn, csq, codebook,
      w_out, b_out[:, None])

    loss = loss_sum[:, 0, 0] / (D * T)
    indices = idx.reshape(B, T)
    return (out, loss, loss, indices, proj)


# speedup vs baseline: 2.8815x; 1.2075x over previous
import jax
import jax.numpy as jnp
from jax.experimental import pallas as pl

B, LATENT, T = 8, 1024, 2048
D, K = 64, 1024
TT = 1024
NT = T // TT


def _vq_kernel(x_ref, w_in_ref, b_in_ref, cbn_ref, csq_ref, cb_ref,
               w_out_ref, b_out_ref,
               out_ref, loss_ref, idx_ref, proj_ref):
    t = pl.program_id(1)

    x = x_ref[0]
    p = jax.lax.dot_general(w_in_ref[...], x, (((1,), (0,)), ((), ())))
    p = p + b_in_ref[...]

    norm = jnp.sqrt(jnp.sum(p * p, axis=0, keepdims=True))
    en = p / jnp.maximum(norm, 1e-12)
    l2 = jnp.sum(en * en, axis=0, keepdims=True)
    s = jax.lax.dot_general(cbn_ref[...], en, (((1,), (0,)), ((), ())))
    dist = l2 - 2.0 * s + csq_ref[...]

    m = jnp.min(dist, axis=0, keepdims=True)
    iota = jax.lax.broadcasted_iota(jnp.int32, dist.shape, 0)
    idx = jnp.min(jnp.where(dist == m, iota, K), axis=0)
    idx_ref[0, 0, :] = idx

    oh = (iota == idx[None, :]).astype(jnp.float32)
    q = jax.lax.dot_general(cb_ref[...], oh, (((0,), (0,)), ((), ())))

    proj_ref[0] = p

    loss_tile = jnp.sum((p - q) ** 2)
    prev = jnp.where(t == 0, jnp.zeros_like(loss_ref), loss_ref[...])
    loss_ref[...] = prev + loss_tile

    out = jax.lax.dot_general(w_out_ref[...], q, (((1,), (0,)), ((), ())))
    out_ref[0] = out + b_out_ref[...]


def kernel(hidden_state, v_in, g_in, b_in, codebook, v_out, g_out, b_out):
    w_in = v_in * (g_in[:, None] / jnp.sqrt(jnp.sum(v_in * v_in, axis=1, keepdims=True)))
    w_out = v_out * (g_out[:, None] / jnp.sqrt(jnp.sum(v_out * v_out, axis=1, keepdims=True)))
    cbn = codebook / jnp.clip(jnp.linalg.norm(codebook, axis=1, keepdims=True), 1e-12)
    csq = jnp.sum(cbn ** 2, axis=1, keepdims=True)

    out_shapes = (
        jax.ShapeDtypeStruct((B, LATENT, T), jnp.float32),
        jax.ShapeDtypeStruct((B, 1, 1), jnp.float32),
        jax.ShapeDtypeStruct((B * NT, 1, TT), jnp.int32),
        jax.ShapeDtypeStruct((B, D, T), jnp.float32),
    )
    out, loss_sum, idx, proj = pl.pallas_call(
        _vq_kernel,
        grid=(B, NT),
        in_specs=[
            pl.BlockSpec((1, LATENT, TT), lambda b, t: (b, 0, t)),
            pl.BlockSpec((D, LATENT), lambda b, t: (0, 0)),
            pl.BlockSpec((D, 1), lambda b, t: (0, 0)),
            pl.BlockSpec((K, D), lambda b, t: (0, 0)),
            pl.BlockSpec((K, 1), lambda b, t: (0, 0)),
            pl.BlockSpec((K, D), lambda b, t: (0, 0)),
            pl.BlockSpec((LATENT, D), lambda b, t: (0, 0)),
            pl.BlockSpec((LATENT, 1), lambda b, t: (0, 0)),
        ],
        out_specs=(
            pl.BlockSpec((1, LATENT, TT), lambda b, t: (b, 0, t)),
            pl.BlockSpec((1, 1, 1), lambda b, t: (b, 0, 0)),
            pl.BlockSpec((1, 1, TT), lambda b, t: (b * NT + t, 0, 0)),
            pl.BlockSpec((1, D, TT), lambda b, t: (b, 0, t)),
        ),
        out_shape=out_shapes,
    )(hidden_state, w_in, b_in[:, None], cbn, csq, codebook,
      w_out, b_out[:, None])

    loss = loss_sum[:, 0, 0] / (D * T)
    indices = idx.reshape(B, T)
    return (out, loss, loss, indices, proj)


# speedup vs baseline: 3.0795x; 1.0687x over previous
import jax
import jax.numpy as jnp
from jax.experimental import pallas as pl

B, LATENT, T = 8, 1024, 2048
D, K = 64, 1024
TT = 2048
NT = T // TT


def _vq_kernel(x_ref, w_in_ref, b_in_ref, cbn_ref, csq_ref, cb_ref,
               w_out_ref, b_out_ref,
               out_ref, loss_ref, idx_ref, proj_ref):
    t = pl.program_id(1)

    x = x_ref[0]
    p = jax.lax.dot_general(w_in_ref[...], x, (((1,), (0,)), ((), ())))
    p = p + b_in_ref[...]

    norm = jnp.sqrt(jnp.sum(p * p, axis=0, keepdims=True))
    en = p / jnp.maximum(norm, 1e-12)
    l2 = jnp.sum(en * en, axis=0, keepdims=True)
    s = jax.lax.dot_general(cbn_ref[...], en, (((1,), (0,)), ((), ())))
    dist = l2 - 2.0 * s + csq_ref[...]

    m = jnp.min(dist, axis=0, keepdims=True)
    iota = jax.lax.broadcasted_iota(jnp.int32, dist.shape, 0)
    idx = jnp.min(jnp.where(dist == m, iota, K), axis=0)
    idx_ref[0, 0, :] = idx

    oh = (iota == idx[None, :]).astype(jnp.float32)
    q = jax.lax.dot_general(cb_ref[...], oh, (((0,), (0,)), ((), ())))

    proj_ref[0] = p

    loss_tile = jnp.sum((p - q) ** 2)
    prev = jnp.where(t == 0, jnp.zeros_like(loss_ref), loss_ref[...])
    loss_ref[...] = prev + loss_tile

    out = jax.lax.dot_general(w_out_ref[...], q, (((1,), (0,)), ((), ())))
    out_ref[0] = out + b_out_ref[...]


def kernel(hidden_state, v_in, g_in, b_in, codebook, v_out, g_out, b_out):
    w_in = v_in * (g_in[:, None] / jnp.sqrt(jnp.sum(v_in * v_in, axis=1, keepdims=True)))
    w_out = v_out * (g_out[:, None] / jnp.sqrt(jnp.sum(v_out * v_out, axis=1, keepdims=True)))
    cbn = codebook / jnp.clip(jnp.linalg.norm(codebook, axis=1, keepdims=True), 1e-12)
    csq = jnp.sum(cbn ** 2, axis=1, keepdims=True)

    out_shapes = (
        jax.ShapeDtypeStruct((B, LATENT, T), jnp.float32),
        jax.ShapeDtypeStruct((B, 1, 1), jnp.float32),
        jax.ShapeDtypeStruct((B * NT, 1, TT), jnp.int32),
        jax.ShapeDtypeStruct((B, D, T), jnp.float32),
    )
    out, loss_sum, idx, proj = pl.pallas_call(
        _vq_kernel,
        grid=(B, NT),
        in_specs=[
            pl.BlockSpec((1, LATENT, TT), lambda b, t: (b, 0, t)),
            pl.BlockSpec((D, LATENT), lambda b, t: (0, 0)),
            pl.BlockSpec((D, 1), lambda b, t: (0, 0)),
            pl.BlockSpec((K, D), lambda b, t: (0, 0)),
            pl.BlockSpec((K, 1), lambda b, t: (0, 0)),
            pl.BlockSpec((K, D), lambda b, t: (0, 0)),
            pl.BlockSpec((LATENT, D), lambda b, t: (0, 0)),
            pl.BlockSpec((LATENT, 1), lambda b, t: (0, 0)),
        ],
        out_specs=(
            pl.BlockSpec((1, LATENT, TT), lambda b, t: (b, 0, t)),
            pl.BlockSpec((1, 1, 1), lambda b, t: (b, 0, 0)),
            pl.BlockSpec((1, 1, TT), lambda b, t: (b * NT + t, 0, 0)),
            pl.BlockSpec((1, D, TT), lambda b, t: (b, 0, t)),
        ),
        out_shape=out_shapes,
    )(hidden_state, w_in, b_in[:, None], cbn, csq, codebook,
      w_out, b_out[:, None])

    loss = loss_sum[:, 0, 0] / (D * T)
    indices = idx.reshape(B, T)
    return (out, loss, loss, indices, proj)


# speedup vs baseline: 3.1955x; 1.0376x over previous
import jax
import jax.numpy as jnp
from jax.experimental import pallas as pl

B, LATENT, T = 8, 1024, 2048
D, K = 64, 1024
TT = 2048
NT = T // TT


def _vq_kernel(x_ref, w_in_ref, b_in_ref, cbn_ref, csq_ref, cb_ref,
               w_out_ref, b_out_ref,
               out_ref, loss_ref, idx_ref, proj_ref):
    t = pl.program_id(1)

    x = x_ref[0]
    p = jax.lax.dot_general(w_in_ref[...], x, (((1,), (0,)), ((), ())))
    p = p + b_in_ref[...]

    norm = jnp.sqrt(jnp.sum(p * p, axis=0, keepdims=True))
    en = p / jnp.maximum(norm, 1e-12)
    l2 = jnp.sum(en * en, axis=0, keepdims=True)
    s = jax.lax.dot_general(cbn_ref[...], en, (((1,), (0,)), ((), ())))
    dist = l2 - 2.0 * s + csq_ref[...]

    idx = jnp.argmin(dist, axis=0)

    idx_ref[0, 0, :] = idx

    iota = jax.lax.broadcasted_iota(jnp.int32, dist.shape, 0)
    oh = (iota == idx[None, :]).astype(jnp.bfloat16)
    q = jax.lax.dot_general(cb_ref[...], oh, (((0,), (0,)), ((), ())),
                            preferred_element_type=jnp.float32)

    proj_ref[0] = p

    loss_tile = jnp.sum((p - q) ** 2)
    prev = jnp.where(t == 0, jnp.zeros_like(loss_ref), loss_ref[...])
    loss_ref[...] = prev + loss_tile

    out = jax.lax.dot_general(w_out_ref[...], q, (((1,), (0,)), ((), ())))
    out_ref[0] = out + b_out_ref[...]


def kernel(hidden_state, v_in, g_in, b_in, codebook, v_out, g_out, b_out):
    w_in = v_in * (g_in[:, None] / jnp.sqrt(jnp.sum(v_in * v_in, axis=1, keepdims=True)))
    w_out = v_out * (g_out[:, None] / jnp.sqrt(jnp.sum(v_out * v_out, axis=1, keepdims=True)))
    cbn = codebook / jnp.clip(jnp.linalg.norm(codebook, axis=1, keepdims=True), 1e-12)
    csq = jnp.sum(cbn ** 2, axis=1, keepdims=True)

    out_shapes = (
        jax.ShapeDtypeStruct((B, LATENT, T), jnp.float32),
        jax.ShapeDtypeStruct((B, 1, 1), jnp.float32),
        jax.ShapeDtypeStruct((B * NT, 1, TT), jnp.int32),
        jax.ShapeDtypeStruct((B, D, T), jnp.float32),
    )
    out, loss_sum, idx, proj = pl.pallas_call(
        _vq_kernel,
        grid=(B, NT),
        in_specs=[
            pl.BlockSpec((1, LATENT, TT), lambda b, t: (b, 0, t)),
            pl.BlockSpec((D, LATENT), lambda b, t: (0, 0)),
            pl.BlockSpec((D, 1), lambda b, t: (0, 0)),
            pl.BlockSpec((K, D), lambda b, t: (0, 0)),
            pl.BlockSpec((K, 1), lambda b, t: (0, 0)),
            pl.BlockSpec((K, D), lambda b, t: (0, 0)),
            pl.BlockSpec((LATENT, D), lambda b, t: (0, 0)),
            pl.BlockSpec((LATENT, 1), lambda b, t: (0, 0)),
        ],
        out_specs=(
            pl.BlockSpec((1, LATENT, TT), lambda b, t: (b, 0, t)),
            pl.BlockSpec((1, 1, 1), lambda b, t: (b, 0, 0)),
            pl.BlockSpec((1, 1, TT), lambda b, t: (b * NT + t, 0, 0)),
            pl.BlockSpec((1, D, TT), lambda b, t: (b, 0, t)),
        ),
        out_shape=out_shapes,
    )(hidden_state, w_in, b_in[:, None], cbn, csq, codebook,
      w_out, b_out[:, None])

    loss = loss_sum[:, 0, 0] / (D * T)
    indices = idx.reshape(B, T)
    return (out, loss, loss, indices, proj)
